# baseline (speedup 1.0000x reference)
import jax
import jax.numpy as jnp
from jax import lax
from jax.experimental import pallas as pl
from jax.experimental.pallas import tpu as pltpu
from jax.experimental.pallas import tpu_sc as plsc

VOCAB_SIZE = 100000
DIM = 1024
MAX_SEQ_LEN = 8192
BATCH = 4
SEQ_LEN = 4096

NC = 2
NS = 16
LANES = 16
NW = NC * NS
POS_PER_W = SEQ_LEN // NW
CHUNK = 32
PCHUNK = POS_PER_W // CHUNK
STEPS = PCHUNK * BATCH
VECS_PER_ROW = DIM // LANES


def _emb_kernel(ids_hbm, table_hbm, pos_hbm, out_hbm,
                idx_v, pos_v, tok0, tok1, gsem0, gsem1, ssem0, ssem1, psem):
    wid = lax.axis_index("s") * NC + lax.axis_index("c")
    base_w = wid * POS_PER_W

    pltpu.sync_copy(ids_hbm.at[wid], idx_v)

    toks = (tok0, tok1)
    gsems = (gsem0, gsem1)
    ssems = (ssem0, ssem1)
    pend_g = [None, None]
    pend_s = [None, None]
    pend_p = None

    pend_g[0] = pltpu.async_copy(table_hbm.at[idx_v.at[0]], tok0, gsem0)

    for s in range(STEPS):
        p, b = divmod(s, BATCH)
        cur = s % 2
        nxt = 1 - cur
        if b == 0:
            pend_p = pltpu.async_copy(
                pos_hbm.at[pl.ds(base_w + p * CHUNK, CHUNK)], pos_v, psem)
        if s + 1 < STEPS:
            if pend_s[nxt] is not None:
                pend_s[nxt].wait()
            pend_g[nxt] = pltpu.async_copy(
                table_hbm.at[idx_v.at[s + 1]], toks[nxt], gsems[nxt])
        pend_g[cur].wait()
        if b == 0:
            pend_p.wait()

        tok = toks[cur]

        def row_body(r, rc, tok=tok):
            for j in range(VECS_PER_ROW):
                sl = pl.ds(j * LANES, LANES)
                plsc.addupdate(tok.at[r, sl], pos_v[r, sl])
            return rc

        lax.fori_loop(0, CHUNK, row_body, 0)

        off = b * SEQ_LEN + base_w + p * CHUNK
        pend_s[cur] = pltpu.async_copy(
            tok, out_hbm.at[pl.ds(off, CHUNK)], ssems[cur])

    pend_s[0].wait()
    pend_s[1].wait()


@jax.jit
def kernel(input_ids, token_embed_weight, pos_embed_weight):
    ids = input_ids.astype(jnp.int32).reshape(BATCH, NW, PCHUNK, CHUNK)
    ids = jnp.transpose(ids, (1, 2, 0, 3)).reshape(NW, STEPS, CHUNK)
    mesh = plsc.VectorSubcoreMesh(core_axis_name="c", subcore_axis_name="s")
    out = pl.kernel(
        _emb_kernel,
        out_type=jax.ShapeDtypeStruct((BATCH * SEQ_LEN, DIM), jnp.float32),
        mesh=mesh,
        scratch_types=[
            pltpu.VMEM((STEPS, CHUNK), jnp.int32),
            pltpu.VMEM((CHUNK, DIM), jnp.float32),
            pltpu.VMEM((CHUNK, DIM), jnp.float32),
            pltpu.VMEM((CHUNK, DIM), jnp.float32),
            pltpu.SemaphoreType.DMA,
            pltpu.SemaphoreType.DMA,
            pltpu.SemaphoreType.DMA,
            pltpu.SemaphoreType.DMA,
            pltpu.SemaphoreType.DMA,
        ],
    )(ids, token_embed_weight, pos_embed_weight)
    return out.reshape(BATCH, SEQ_LEN, DIM)


# speedup vs baseline: 1.6337x; 1.6337x over previous
import jax
import jax.numpy as jnp
from jax import lax
from jax.experimental import pallas as pl
from jax.experimental.pallas import tpu as pltpu
from jax.experimental.pallas import tpu_sc as plsc

VOCAB_SIZE = 100000
DIM = 1024
MAX_SEQ_LEN = 8192
BATCH = 4
SEQ_LEN = 4096

NC = 2
NS = 16
LANES = 16
NW = NC * NS
POS_PER_W = SEQ_LEN // NW
CHUNK = 32
PCHUNK = POS_PER_W // CHUNK
STEPS = PCHUNK * BATCH
VECS_PER_ROW = DIM // LANES


def _emb_kernel(ids_hbm, table_hbm, pos_hbm, out_hbm,
                idx_v, idx0, idx1, pos_v, tok0, tok1,
                gsem0, gsem1, ssem0, ssem1, psem):
    wid = lax.axis_index("s") * NC + lax.axis_index("c")
    base_w = wid * POS_PER_W

    pltpu.sync_copy(ids_hbm.at[wid], idx_v)

    toks = (tok0, tok1)
    idxs = (idx0, idx1)
    gsems = (gsem0, gsem1)
    ssems = (ssem0, ssem1)
    pend_g = [None, None]
    pend_s = [None, None]
    pend_p = None

    def prep(t, buf):
        for v in range(CHUNK // LANES):
            sl = pl.ds(v * LANES, LANES)
            idxs[buf][sl] = idx_v[t, sl]
        pend_g[buf] = pltpu.async_copy(
            table_hbm.at[idxs[buf]], toks[buf], gsems[buf])

    prep(0, 0)
    pend_p = pltpu.async_copy(pos_hbm.at[pl.ds(base_w, CHUNK)], pos_v, psem)
    for s in range(STEPS):
        p, b = divmod(s, BATCH)
        cur = s % 2
        nxt = 1 - cur
        if s + 1 < STEPS:
            if pend_s[nxt] is not None:
                pend_s[nxt].wait()
            prep(s + 1, nxt)
        pend_g[cur].wait()
        if b == 0:
            pend_p.wait()

        tok = toks[cur]

        @plsc.parallel_loop(0, CHUNK)
        def _row(r, tok=tok):
            for j in range(VECS_PER_ROW):
                sl = pl.ds(j * LANES, LANES)
                tok[r, sl] = tok[r, sl] + pos_v[r, sl]

        off = b * SEQ_LEN + base_w + p * CHUNK
        pend_s[cur] = pltpu.async_copy(
            tok, out_hbm.at[pl.ds(off, CHUNK)], ssems[cur])
        if b == BATCH - 1 and p + 1 < PCHUNK:
            pend_p = pltpu.async_copy(
                pos_hbm.at[pl.ds(base_w + (p + 1) * CHUNK, CHUNK)],
                pos_v, psem)

    pend_s[0].wait()
    pend_s[1].wait()


@jax.jit
def kernel(input_ids, token_embed_weight, pos_embed_weight):
    ids = input_ids.astype(jnp.int32).reshape(BATCH, NW, PCHUNK, CHUNK)
    ids = jnp.transpose(ids, (1, 2, 0, 3)).reshape(NW, STEPS, CHUNK)
    mesh = plsc.VectorSubcoreMesh(core_axis_name="c", subcore_axis_name="s")
    out = pl.kernel(
        _emb_kernel,
        out_type=jax.ShapeDtypeStruct((BATCH * SEQ_LEN, DIM), jnp.float32),
        mesh=mesh,
        scratch_types=[
            pltpu.VMEM((STEPS, CHUNK), jnp.int32),
            pltpu.VMEM((CHUNK,), jnp.int32),
            pltpu.VMEM((CHUNK,), jnp.int32),
            pltpu.VMEM((CHUNK, DIM), jnp.float32),
            pltpu.VMEM((CHUNK, DIM), jnp.float32),
            pltpu.VMEM((CHUNK, DIM), jnp.float32),
            pltpu.SemaphoreType.DMA,
            pltpu.SemaphoreType.DMA,
            pltpu.SemaphoreType.DMA,
            pltpu.SemaphoreType.DMA,
            pltpu.SemaphoreType.DMA,
        ],
    )(ids, token_embed_weight, pos_embed_weight)
    return out.reshape(BATCH, SEQ_LEN, DIM)


# speedup vs baseline: 2.0375x; 1.2471x over previous
import jax
import jax.numpy as jnp
from jax import lax
from jax.experimental import pallas as pl
from jax.experimental.pallas import tpu as pltpu
from jax.experimental.pallas import tpu_sc as plsc

VOCAB_SIZE = 100000
DIM = 1024
MAX_SEQ_LEN = 8192
BATCH = 4
SEQ_LEN = 4096

NC = 2
NS = 16
LANES = 16
NW = NC * NS
POS_PER_W = SEQ_LEN // NW
CHUNK = 16
PCHUNK = POS_PER_W // CHUNK
STEPS = PCHUNK * BATCH
VECS_PER_ROW = DIM // LANES
NBUF = 4
LOOKAHEAD = 2
INNER = 2 * BATCH
OUTER = STEPS // INNER


def _emb_kernel(ids_hbm, table_hbm, pos_hbm, out_hbm,
                idx_v, idx0, idx1, idx2, idx3, pos0, pos1,
                tok0, tok1, tok2, tok3,
                g0, g1, g2, g3, s0, s1, s2, s3, p0, p1):
    wid = lax.axis_index("s") * NC + lax.axis_index("c")
    base_w = wid * POS_PER_W

    toks = (tok0, tok1, tok2, tok3)
    idxs = (idx0, idx1, idx2, idx3)
    gsems = (g0, g1, g2, g3)
    ssems = (s0, s1, s2, s3)
    poss = (pos0, pos1)
    psems = (p0, p1)

    pltpu.sync_copy(ids_hbm.at[wid], idx_v)

    def issue_gather(t, k):
        idxs[k][pl.ds(0, LANES)] = idx_v[t, pl.ds(0, LANES)]
        return pltpu.async_copy(table_hbm.at[idxs[k]], toks[k], gsems[k])

    def gather_wait(k):
        pltpu.make_async_copy(table_hbm.at[idxs[k]], toks[k], gsems[k]).wait()

    def store(t_p, t_b, k):
        off = t_b * SEQ_LEN + base_w + t_p * CHUNK
        return pltpu.async_copy(toks[k], out_hbm.at[pl.ds(off, CHUNK)],
                                ssems[k])

    def store_wait(k):
        pltpu.make_async_copy(toks[k], out_hbm.at[pl.ds(base_w, CHUNK)],
                              ssems[k]).wait()

    def pos_load(p, pb):
        return pltpu.async_copy(pos_hbm.at[pl.ds(base_w + p * CHUNK, CHUNK)],
                                poss[pb], psems[pb])

    def pos_wait(pb):
        pltpu.make_async_copy(pos_hbm.at[pl.ds(base_w, CHUNK)], poss[pb],
                              psems[pb]).wait()

    pos_load(0, 0)
    issue_gather(0, 0)
    issue_gather(1, 1)

    def outer_body(o, carry):
        s_base = o * INNER
        for j in range(INNER):
            s = s_base + j
            k = j % NBUF
            k2 = (j + LOOKAHEAD) % NBUF
            pb = 0 if j < BATCH else 1
            b = j % BATCH
            p = 2 * o + (j // BATCH)

            if j == 0:
                pos_load(2 * o + 1, 1)
            if j == BATCH:
                @pl.when(o + 1 < OUTER)
                def _():
                    pos_load(2 * o + 2, 0)

            if j < INNER - LOOKAHEAD or True:
                s2 = s + LOOKAHEAD

                @pl.when(s2 < STEPS)
                def _():
                    @pl.when(s2 >= NBUF)
                    def _():
                        store_wait(k2)
                    issue_gather(s2, k2)

            gather_wait(k)
            if j == 0 or j == BATCH:
                pos_wait(pb)

            tok = toks[k]
            posb = poss[pb]

            @plsc.parallel_loop(0, CHUNK)
            def _row(r, tok=tok, posb=posb):
                for v in range(VECS_PER_ROW):
                    sl = pl.ds(v * LANES, LANES)
                    tok[r, sl] = tok[r, sl] + posb[r, sl]

            store(p, b, k)
        return carry

    lax.fori_loop(0, OUTER, outer_body, 0)

    for k in range(NBUF):
        store_wait(k)


@jax.jit
def kernel(input_ids, token_embed_weight, pos_embed_weight):
    ids = input_ids.astype(jnp.int32).reshape(BATCH, NW, PCHUNK, CHUNK)
    ids = jnp.transpose(ids, (1, 2, 0, 3)).reshape(NW, STEPS, CHUNK)
    mesh = plsc.VectorSubcoreMesh(core_axis_name="c", subcore_axis_name="s")
    out = pl.kernel(
        _emb_kernel,
        out_type=jax.ShapeDtypeStruct((BATCH * SEQ_LEN, DIM), jnp.float32),
        mesh=mesh,
        scratch_types=[
            pltpu.VMEM((STEPS, CHUNK), jnp.int32),
            pltpu.VMEM((CHUNK,), jnp.int32),
            pltpu.VMEM((CHUNK,), jnp.int32),
            pltpu.VMEM((CHUNK,), jnp.int32),
            pltpu.VMEM((CHUNK,), jnp.int32),
            pltpu.VMEM((CHUNK, DIM), jnp.float32),
            pltpu.VMEM((CHUNK, DIM), jnp.float32),
            pltpu.VMEM((CHUNK, DIM), jnp.float32),
            pltpu.VMEM((CHUNK, DIM), jnp.float32),
            pltpu.VMEM((CHUNK, DIM), jnp.float32),
            pltpu.VMEM((CHUNK, DIM), jnp.float32),
            pltpu.SemaphoreType.DMA,
            pltpu.SemaphoreType.DMA,
            pltpu.SemaphoreType.DMA,
            pltpu.SemaphoreType.DMA,
            pltpu.SemaphoreType.DMA,
            pltpu.SemaphoreType.DMA,
            pltpu.SemaphoreType.DMA,
            pltpu.SemaphoreType.DMA,
            pltpu.SemaphoreType.DMA,
            pltpu.SemaphoreType.DMA,
        ],
    )(ids, token_embed_weight, pos_embed_weight)
    return out.reshape(BATCH, SEQ_LEN, DIM)
